# baseline (speedup 1.0000x reference)
import functools

import jax
import jax.numpy as jnp
from jax import lax
from jax.experimental import pallas as pl
from jax.experimental.pallas import tpu as pltpu
from jax.experimental.pallas import tpu_sc as plsc

NUM_EMB = 1000000
DIM = 64
WPR = DIM // 4
B = 16384 * 26
NW = 32
PER_W = B // NW
CHUNK = 1024
NCHUNK = PER_W // CHUNK
SUB = 128
NSUB = CHUNK // SUB
GROUPS = CHUNK // 16

_SCALE_C = float(2.0 ** -112) / 127.0


def _dequant_group(idx_v, scl_v, rows_v, out_v, r0, iota2):
    iv = idx_v[pl.ds(r0, 16)]
    sv = scl_v[pl.ds(r0, 16)]
    sev = jnp.where(iv == 0, jnp.float32(0.0), sv) * jnp.float32(_SCALE_C)
    for rr in range(16):
        se = sev[rr]
        w = rows_v[r0 + rr]
        hs = []
        for k in range(4):
            if k == 3:
                bk = lax.shift_right_arithmetic(w, 24)
            else:
                bk = lax.shift_right_arithmetic(
                    lax.shift_left(w, 24 - 8 * k), 24)
            p = bk.astype(jnp.float32) * se
            bits = lax.bitcast_convert_type(p, jnp.int32)
            mag = lax.bitwise_and(bits, jnp.int32(0x7FFFFFFF))
            hm = lax.shift_right_logical(mag + jnp.int32(0xFFF), 13)
            sg = lax.bitwise_and(lax.shift_right_logical(bits, 16),
                                 jnp.int32(0x8000))
            hs.append(lax.bitwise_or(hm, sg))
        we = lax.bitwise_or(hs[0], lax.shift_left(hs[1], 16))
        wo = lax.bitwise_or(hs[2], lax.shift_left(hs[3], 16))
        obase = (r0 + rr) * (2 * WPR) + iota2
        plsc.store_scatter(out_v, [obase], we)
        plsc.store_scatter(out_v, [obase + 1], wo)


def _sc_body(idx_hbm, tab_hbm, scl_hbm, out_hbm,
             idx_v, scl_v, rows_v, out_v, sem):
    cid = lax.axis_index("c")
    sid = lax.axis_index("s")
    wid = sid * 2 + cid
    tbase = wid * PER_W
    iota2 = lax.iota(jnp.int32, 16) * 2

    def chunk_body(k, carry):
        base = tbase + k * CHUNK
        pltpu.sync_copy(idx_hbm.at[pl.ds(base, CHUNK)], idx_v)
        copies = []
        for j in range(NSUB):
            s = pl.ds(j * SUB, SUB)
            copies.append(
                pltpu.async_copy(tab_hbm.at[idx_v.at[s]], rows_v.at[s], sem))
            copies.append(
                pltpu.async_copy(scl_hbm.at[idx_v.at[s]], scl_v.at[s], sem))
        for cp in copies:
            cp.wait()

        def group_body(g, c2):
            _dequant_group(idx_v, scl_v, rows_v, out_v, g * 16, iota2)
            return c2

        lax.fori_loop(0, GROUPS, group_body, 0)
        pltpu.sync_copy(out_v,
                        out_hbm.at[pl.ds(base * 2 * WPR, CHUNK * 2 * WPR)])
        return carry

    lax.fori_loop(0, NCHUNK, chunk_body, 0)


@jax.jit
def _sc_lookup(idx, tab32, scales):
    mesh = plsc.VectorSubcoreMesh(core_axis_name="c", subcore_axis_name="s",
                                  num_cores=2, num_subcores=16)
    f = pl.kernel(
        _sc_body,
        out_type=jax.ShapeDtypeStruct((B * 2 * WPR,), jnp.int32),
        mesh=mesh,
        scratch_types=[
            pltpu.VMEM((CHUNK,), jnp.int32),
            pltpu.VMEM((CHUNK,), jnp.float32),
            pltpu.VMEM((CHUNK, WPR), jnp.int32),
            pltpu.VMEM((CHUNK * 2 * WPR,), jnp.int32),
            pltpu.SemaphoreType.DMA,
        ],
        compiler_params=pltpu.CompilerParams(needs_layout_passes=False,
                                             use_tc_tiling_on_sc=False),
    )
    return f(idx, tab32, scales)


def kernel(input, weight_int8, weight_scales):
    idx = input.reshape(-1).astype(jnp.int32)
    tab32 = lax.bitcast_convert_type(
        weight_int8.reshape(NUM_EMB, WPR, 4), jnp.int32)
    out32 = _sc_lookup(idx, tab32, weight_scales)
    out = lax.bitcast_convert_type(out32, jnp.float16)
    return out.reshape(input.shape + (DIM,))
